# Initial kernel scaffold; baseline (speedup 1.0000x reference)
import jax
import jax.numpy as jnp
from jax.experimental import pallas as pl


def kernel(protein_n_id, go_term_n_id, go_term_x, e_gp_src, e_gp_dst, e_pg_src, e_pg_dst, label_src, label_dst, protein_emb, go_term_emb, lin_W, lin_b, Wl, bl, Wr):
    raise NotImplementedError("write your pallas kernel here")


# speedup vs baseline: 7.0720x; 7.0720x over previous
import functools

import jax
import jax.numpy as jnp
from jax import lax
from jax.experimental import pallas as pl
from jax.experimental.pallas import tpu as pltpu
from jax.experimental.pallas import tpu_sc as plsc

N_P, N_G, D, E, L = 50000, 10000, 64, 800000, 100000

NC, NS = 2, 16
HALF = 25088
P_PAD = 2 * HALF
E_CH = 512
E_PAD = 802816
L_TILE = 3136
L_CH = 448
L_PAD = 32 * L_TILE

_MESH = plsc.VectorSubcoreMesh(
    core_axis_name="c", subcore_axis_name="s", num_cores=NC, num_subcores=NS)
_SC_PARAMS = pltpu.CompilerParams(
    use_tc_tiling_on_sc=False, needs_layout_passes=False)


def _iota16():
  return lax.iota(jnp.int32, 16)


def _acc_rows(own):
  return -(-(own + 16) // 128) * 128


def _out_rows(own):
  r = own // 16
  return own if r % 8 == 0 else _acc_rows(own)


def _make_agg(mode):
  if mode == "p":
    own, edges_per_tile, ch = HALF, E_PAD // NS, 256
  else:
    own, edges_per_tile, ch = N_G, E_PAD // (NC * NS), E_CH
  acc_rows = _acc_rows(own)
  out_rows = _out_rows(own)
  n_chunks = edges_per_tile // ch
  zrows = acc_rows // 16
  orows = out_rows // 16

  def body(table, srcp, dstp, zinit, out, sidx, didx, dloc, rows, acc):
    c = lax.axis_index("c")
    s = lax.axis_index("s")
    z0 = pl.multiple_of(s * zrows, zrows)
    pltpu.sync_copy(zinit.at[pl.ds(z0, zrows)], acc.at[pl.ds(z0, zrows)])
    plsc.subcore_barrier()
    if mode == "p":
      ebase = s * edges_per_tile
      row_base = c * HALF
    else:
      ebase = (c * NS + s) * edges_per_tile
      row_base = 0
    tr = own + _iota16()

    def chunk(i, carry):
      eb = pl.multiple_of(ebase + i * ch, ch)
      pltpu.sync_copy(srcp.at[pl.ds(eb, ch)], sidx)
      pltpu.sync_copy(dstp.at[pl.ds(eb, ch)], didx)
      pltpu.sync_copy(table.at[sidx], rows)
      for j in range(ch // 16):
        d = didx[pl.ds(j * 16, 16)]
        loc = d - row_base
        ok = (loc >= 0) & (loc < own)
        dloc[pl.ds(j * 16, 16)] = jnp.where(ok, loc, tr)
      pltpu.sync_copy(rows, acc.at[dloc], add=True)
      return carry

    lax.fori_loop(0, n_chunks, chunk, 0)
    plsc.subcore_barrier()
    o0 = pl.multiple_of(s * orows, orows)
    pltpu.sync_copy(acc.at[pl.ds(o0, orows)], out.at[c, pl.ds(o0, orows)])

  return pl.kernel(
      body,
      out_type=jax.ShapeDtypeStruct((NC, out_rows, D), jnp.float32),
      mesh=_MESH,
      compiler_params=_SC_PARAMS,
      scratch_types=[
          pltpu.VMEM((ch,), jnp.int32),
          pltpu.VMEM((ch,), jnp.int32),
          pltpu.VMEM((ch,), jnp.int32),
          pltpu.VMEM((ch, D), jnp.float32),
          pltpu.VMEM_SHARED((acc_rows, D), jnp.float32),
      ],
  )


def _make_cnt(mode):
  if mode == "p":
    own, edges_per_tile = HALF, E_PAD // NS
  else:
    own, edges_per_tile = N_G, E_PAD // (NC * NS)
  acc_rows = _acc_rows(own)
  out_rows = _out_rows(own)
  n_chunks = edges_per_tile // E_CH
  zrows = acc_rows // 16
  orows = out_rows // 16

  def body(dstp, zinit, ones, out, didx, dloc, ones_v, acc):
    c = lax.axis_index("c")
    s = lax.axis_index("s")
    z0 = pl.multiple_of(s * zrows, zrows)
    pltpu.sync_copy(zinit.at[pl.ds(z0, zrows)], acc.at[pl.ds(z0, zrows)])
    pltpu.sync_copy(ones, ones_v)
    plsc.subcore_barrier()
    if mode == "p":
      ebase = s * edges_per_tile
      row_base = c * HALF
    else:
      ebase = (c * NS + s) * edges_per_tile
      row_base = 0
    tr = own + _iota16()

    def chunk(i, carry):
      eb = pl.multiple_of(ebase + i * E_CH, E_CH)
      pltpu.sync_copy(dstp.at[pl.ds(eb, E_CH)], didx)
      for j in range(E_CH // 16):
        d = didx[pl.ds(j * 16, 16)]
        loc = d - row_base
        ok = (loc >= 0) & (loc < own)
        dloc[pl.ds(j * 16, 16)] = jnp.where(ok, loc, tr)
      pltpu.sync_copy(ones_v, acc.at[dloc], add=True)
      return carry

    lax.fori_loop(0, n_chunks, chunk, 0)
    plsc.subcore_barrier()
    o0 = pl.multiple_of(s * orows, orows)
    pltpu.sync_copy(acc.at[pl.ds(o0, orows)], out.at[c, pl.ds(o0, orows)])

  return pl.kernel(
      body,
      out_type=jax.ShapeDtypeStruct((NC, out_rows, 8), jnp.float32),
      mesh=_MESH,
      compiler_params=_SC_PARAMS,
      scratch_types=[
          pltpu.VMEM((E_CH,), jnp.int32),
          pltpu.VMEM((E_CH,), jnp.int32),
          pltpu.VMEM((E_CH, 8), jnp.float32),
          pltpu.VMEM_SHARED((acc_rows, 8), jnp.float32),
      ],
  )


def _cls_body(xp, xg, ls, ld, out, sidx, didx, rp, rg, tb, ov):
  c = lax.axis_index("c")
  s = lax.axis_index("s")
  base = (c * NS + s) * L_TILE
  cols = [_iota16() + 16 * m for m in range(4)]
  ridx = _iota16() * 16

  for k in range(L_TILE // L_CH):
    cb = pl.multiple_of(base + k * L_CH, L_CH)
    pltpu.sync_copy(ls.at[pl.ds(cb, L_CH)], sidx)
    pltpu.sync_copy(ld.at[pl.ds(cb, L_CH)], didx)
    pltpu.sync_copy(xp.at[sidx], rp)
    pltpu.sync_copy(xg.at[didx], rg)

    def g16(g, carry):
      for j in range(16):
        r = jnp.full((16,), g * 16 + j, jnp.int32)
        acc = None
        for m in range(4):
          a = plsc.load_gather(rp, [r, cols[m]])
          b = plsc.load_gather(rg, [r, cols[m]])
          ab = a * b
          acc = ab if acc is None else acc + ab
        tb[pl.ds(j * 16, 16)] = acc
      tot = jnp.zeros((16,), jnp.float32)
      for m in range(16):
        tot = tot + plsc.load_gather(tb, [ridx + m])
      ov[pl.ds(g * 16, 16)] = tot
      return carry

    lax.fori_loop(0, L_CH // 16, g16, 0)
    pltpu.sync_copy(ov, out.at[pl.ds(cb, L_CH)])


_classifier = pl.kernel(
    _cls_body,
    out_type=jax.ShapeDtypeStruct((L_PAD,), jnp.float32),
    mesh=_MESH,
    compiler_params=_SC_PARAMS,
    scratch_types=[
        pltpu.VMEM((L_CH,), jnp.int32),
        pltpu.VMEM((L_CH,), jnp.int32),
        pltpu.VMEM((L_CH, D), jnp.float32),
        pltpu.VMEM((L_CH, D), jnp.float32),
        pltpu.VMEM((256,), jnp.float32),
        pltpu.VMEM((L_CH,), jnp.float32),
    ],
)


def _init_xg_body(gx, w, b, ge, out):
  acc = lax.dot_general(gx[...], w[...], (((1,), (1,)), ((), ())),
                        preferred_element_type=jnp.float32)
  out[...] = acc + b[...] + ge[...]


def _init_xg(gx, w, b2, ge):
  blk = 1000
  return pl.pallas_call(
      _init_xg_body,
      grid=(N_G // blk,),
      in_specs=[
          pl.BlockSpec((blk, 1000), lambda i: (i, 0)),
          pl.BlockSpec((D, 1000), lambda i: (0, 0)),
          pl.BlockSpec((1, D), lambda i: (0, 0)),
          pl.BlockSpec((blk, D), lambda i: (i, 0)),
      ],
      out_specs=pl.BlockSpec((blk, D), lambda i: (i, 0)),
      out_shape=jax.ShapeDtypeStruct((N_G, D), jnp.float32),
  )(gx, w, b2, ge)


def _make_transform_body(planes, relu):
  def body(x, agg, cnt, wl, wr, b, out):
    a = agg[0]
    n = cnt[0, :, 0:1]
    for p in range(1, planes):
      a = a + agg[p]
      n = n + cnt[p, :, 0:1]
    mean = a / jnp.maximum(n, 1.0)
    o = (lax.dot_general(mean, wl[...], (((1,), (1,)), ((), ())),
                         preferred_element_type=jnp.float32)
         + lax.dot_general(x[...], wr[...], (((1,), (1,)), ((), ())),
                           preferred_element_type=jnp.float32)
         + b[...])
    if relu:
      o = jnp.maximum(o, 0.0)
    out[...] = o
  return body


def _transform(x, agg, cnt, wl, wr, b2, relu, blk):
  planes = agg.shape[0]
  rows = x.shape[0]
  return pl.pallas_call(
      _make_transform_body(planes, relu),
      grid=(rows // blk,),
      in_specs=[
          pl.BlockSpec((blk, D), lambda i: (i, 0)),
          pl.BlockSpec((planes, blk, D), lambda i: (0, i, 0)),
          pl.BlockSpec((planes, blk, 8), lambda i: (0, i, 0)),
          pl.BlockSpec((D, D), lambda i: (0, 0)),
          pl.BlockSpec((D, D), lambda i: (0, 0)),
          pl.BlockSpec((1, D), lambda i: (0, 0)),
      ],
      out_specs=pl.BlockSpec((blk, D), lambda i: (i, 0)),
      out_shape=jax.ShapeDtypeStruct((rows, D), jnp.float32),
  )(x, agg, cnt, wl, wr, b2)


_agg_p = _make_agg("p")
_agg_g = _make_agg("g")
_cnt_p = _make_cnt("p")
_cnt_g = _make_cnt("g")


def kernel(protein_n_id, go_term_n_id, go_term_x, e_gp_src, e_gp_dst,
           e_pg_src, e_pg_dst, label_src, label_dst, protein_emb,
           go_term_emb, lin_W, lin_b, Wl, bl, Wr):
  f32 = jnp.float32
  xp = jnp.concatenate(
      [protein_emb, jnp.zeros((P_PAD - N_P, D), f32)], axis=0)
  xg = _init_xg(go_term_x, lin_W, lin_b.reshape(1, D), go_term_emb)

  epad = E_PAD - E
  zpad_i = jnp.zeros((epad,), jnp.int32)
  npad_i = jnp.full((epad,), -1, jnp.int32)
  gp_s = jnp.concatenate([e_gp_src, zpad_i])
  gp_d = jnp.concatenate([e_gp_dst, npad_i])
  pg_s = jnp.concatenate([e_pg_src, zpad_i])
  pg_d = jnp.concatenate([e_pg_dst, npad_i])

  z64_p = jnp.zeros((_acc_rows(HALF), D), f32)
  z64_g = jnp.zeros((_acc_rows(N_G), D), f32)
  z8_p = jnp.zeros((_acc_rows(HALF), 8), f32)
  z8_g = jnp.zeros((_acc_rows(N_G), 8), f32)
  ones8 = jnp.ones((E_CH, 8), f32)

  cnt_p = _cnt_p(gp_d, z8_p, ones8).reshape(1, P_PAD, 8)
  cnt_g = _cnt_g(pg_d, z8_g, ones8)

  for layer in range(3):
    relu = layer < 2
    agg_p = _agg_p(xg, gp_s, gp_d, z64_p).reshape(1, P_PAD, D)
    agg_g = _agg_g(xp, pg_s, pg_d, z64_g)
    xp = _transform(xp, agg_p, cnt_p, Wl[2 * layer], Wr[2 * layer],
                    bl[2 * layer].reshape(1, D), relu, 512)
    xg = _transform(xg, agg_g, cnt_g, Wl[2 * layer + 1], Wr[2 * layer + 1],
                    bl[2 * layer + 1].reshape(1, D), relu, 1000)

  lpad = L_PAD - L
  ls = jnp.concatenate([label_src, jnp.zeros((lpad,), jnp.int32)])
  ld = jnp.concatenate([label_dst, jnp.zeros((lpad,), jnp.int32)])
  pred = _classifier(xp, xg, ls, ld)
  return pred[:L]


# speedup vs baseline: 9.2846x; 1.3129x over previous
import functools

import jax
import jax.numpy as jnp
from jax import lax
from jax.experimental import pallas as pl
from jax.experimental.pallas import tpu as pltpu
from jax.experimental.pallas import tpu_sc as plsc

N_P, N_G, D, E, L = 50000, 10000, 64, 800000, 100000

NC, NS = 2, 16
HALF = 25088
P_PAD = 2 * HALF
E_CH = 512
E_PAD = 802816
L_TILE = 3136
L_CH = 448
L_PAD = 32 * L_TILE

_MESH = plsc.VectorSubcoreMesh(
    core_axis_name="c", subcore_axis_name="s", num_cores=NC, num_subcores=NS)
_SC_PARAMS = pltpu.CompilerParams(
    use_tc_tiling_on_sc=False, needs_layout_passes=False)


def _iota16():
  return lax.iota(jnp.int32, 16)


def _acc_rows(own):
  return -(-(own + 16) // 128) * 128


def _out_rows(own):
  r = own // 16
  return own if r % 8 == 0 else _acc_rows(own)


def _make_agg(mode):
  if mode == "p":
    own, edges_per_tile, ch = HALF, E_PAD // NS, 224
  else:
    own, edges_per_tile, ch = N_G, E_PAD // (NC * NS), 448
  acc_rows = _acc_rows(own)
  out_rows = _out_rows(own)
  n_chunks = edges_per_tile // ch
  assert n_chunks % 2 == 0
  zrows = acc_rows // 16
  orows = out_rows // 16

  def body(table, srcp, dstp, zinit, out,
           sidx0, sidx1, didx0, didx1, rows0, rows1, sem0, sem1, acc):
    c = lax.axis_index("c")
    s = lax.axis_index("s")
    z0 = pl.multiple_of(s * zrows, zrows)
    pltpu.sync_copy(zinit.at[pl.ds(z0, zrows)], acc.at[pl.ds(z0, zrows)])
    plsc.subcore_barrier()
    if mode == "p":
      ebase = s * edges_per_tile
      row_base = c * HALF
    else:
      ebase = (c * NS + s) * edges_per_tile
      row_base = 0
    tr = own + _iota16()
    sidx = (sidx0, sidx1)
    didx = (didx0, didx1)
    rows = (rows0, rows1)
    sem = (sem0, sem1)

    def load(i, b):
      eb = pl.multiple_of(ebase + i * ch, 32)
      pltpu.sync_copy(srcp.at[pl.ds(eb, ch)], sidx[b])
      pltpu.sync_copy(dstp.at[pl.ds(eb, ch)], didx[b])
      pltpu.async_copy(table.at[sidx[b]], rows[b], sem[b])

    load(0, 0)

    def pair(i2, carry):
      i = i2 * 2
      for b in (0, 1):
        @pl.when(i + b + 1 < n_chunks)
        def _():
          load(i + b + 1, 1 - b)
        for j in range(ch // 16):
          d = didx[b][pl.ds(j * 16, 16)]
          loc = d - row_base
          ok = (loc >= 0) & (loc < own)
          didx[b][pl.ds(j * 16, 16)] = jnp.where(ok, loc, tr)
        pltpu.make_async_copy(table.at[sidx[b]], rows[b], sem[b]).wait()
        pltpu.sync_copy(rows[b], acc.at[didx[b]], add=True)
      return carry

    lax.fori_loop(0, n_chunks // 2, pair, 0)
    plsc.subcore_barrier()
    o0 = pl.multiple_of(s * orows, orows)
    pltpu.sync_copy(acc.at[pl.ds(o0, orows)], out.at[c, pl.ds(o0, orows)])

  return pl.kernel(
      body,
      out_type=jax.ShapeDtypeStruct((NC, out_rows, D), jnp.float32),
      mesh=_MESH,
      compiler_params=_SC_PARAMS,
      scratch_types=[
          pltpu.VMEM((ch,), jnp.int32),
          pltpu.VMEM((ch,), jnp.int32),
          pltpu.VMEM((ch,), jnp.int32),
          pltpu.VMEM((ch,), jnp.int32),
          pltpu.VMEM((ch, D), jnp.float32),
          pltpu.VMEM((ch, D), jnp.float32),
          pltpu.SemaphoreType.DMA,
          pltpu.SemaphoreType.DMA,
          pltpu.VMEM_SHARED((acc_rows, D), jnp.float32),
      ],
  )


def _make_cnt(mode):
  if mode == "p":
    own, edges_per_tile = HALF, E_PAD // NS
  else:
    own, edges_per_tile = N_G, E_PAD // (NC * NS)
  acc_rows = _acc_rows(own)
  out_rows = _out_rows(own)
  n_chunks = edges_per_tile // E_CH
  zrows = acc_rows // 16
  orows = out_rows // 16

  def body(dstp, zinit, ones, out, didx, dloc, ones_v, acc):
    c = lax.axis_index("c")
    s = lax.axis_index("s")
    z0 = pl.multiple_of(s * zrows, zrows)
    pltpu.sync_copy(zinit.at[pl.ds(z0, zrows)], acc.at[pl.ds(z0, zrows)])
    pltpu.sync_copy(ones, ones_v)
    plsc.subcore_barrier()
    if mode == "p":
      ebase = s * edges_per_tile
      row_base = c * HALF
    else:
      ebase = (c * NS + s) * edges_per_tile
      row_base = 0
    tr = own + _iota16()

    def chunk(i, carry):
      eb = pl.multiple_of(ebase + i * E_CH, E_CH)
      pltpu.sync_copy(dstp.at[pl.ds(eb, E_CH)], didx)
      for j in range(E_CH // 16):
        d = didx[pl.ds(j * 16, 16)]
        loc = d - row_base
        ok = (loc >= 0) & (loc < own)
        dloc[pl.ds(j * 16, 16)] = jnp.where(ok, loc, tr)
      pltpu.sync_copy(ones_v, acc.at[dloc], add=True)
      return carry

    lax.fori_loop(0, n_chunks, chunk, 0)
    plsc.subcore_barrier()
    o0 = pl.multiple_of(s * orows, orows)
    pltpu.sync_copy(acc.at[pl.ds(o0, orows)], out.at[c, pl.ds(o0, orows)])

  return pl.kernel(
      body,
      out_type=jax.ShapeDtypeStruct((NC, out_rows, 8), jnp.float32),
      mesh=_MESH,
      compiler_params=_SC_PARAMS,
      scratch_types=[
          pltpu.VMEM((E_CH,), jnp.int32),
          pltpu.VMEM((E_CH,), jnp.int32),
          pltpu.VMEM((E_CH, 8), jnp.float32),
          pltpu.VMEM_SHARED((acc_rows, 8), jnp.float32),
      ],
  )


def _cls_body(xp, xg, ls, ld, out, sidx, didx, rp, rg, tb, ov):
  c = lax.axis_index("c")
  s = lax.axis_index("s")
  base = (c * NS + s) * L_TILE
  cols = [_iota16() + 16 * m for m in range(4)]
  ridx = _iota16() * 16

  for k in range(L_TILE // L_CH):
    cb = pl.multiple_of(base + k * L_CH, L_CH)
    pltpu.sync_copy(ls.at[pl.ds(cb, L_CH)], sidx)
    pltpu.sync_copy(ld.at[pl.ds(cb, L_CH)], didx)
    pltpu.sync_copy(xp.at[sidx], rp)
    pltpu.sync_copy(xg.at[didx], rg)

    def g16(g, carry):
      for j in range(16):
        r = jnp.full((16,), g * 16 + j, jnp.int32)
        acc = None
        for m in range(4):
          a = plsc.load_gather(rp, [r, cols[m]])
          b = plsc.load_gather(rg, [r, cols[m]])
          ab = a * b
          acc = ab if acc is None else acc + ab
        tb[pl.ds(j * 16, 16)] = acc
      tot = jnp.zeros((16,), jnp.float32)
      for m in range(16):
        tot = tot + plsc.load_gather(tb, [ridx + m])
      ov[pl.ds(g * 16, 16)] = tot
      return carry

    lax.fori_loop(0, L_CH // 16, g16, 0)
    pltpu.sync_copy(ov, out.at[pl.ds(cb, L_CH)])


_classifier = pl.kernel(
    _cls_body,
    out_type=jax.ShapeDtypeStruct((L_PAD,), jnp.float32),
    mesh=_MESH,
    compiler_params=_SC_PARAMS,
    scratch_types=[
        pltpu.VMEM((L_CH,), jnp.int32),
        pltpu.VMEM((L_CH,), jnp.int32),
        pltpu.VMEM((L_CH, D), jnp.float32),
        pltpu.VMEM((L_CH, D), jnp.float32),
        pltpu.VMEM((256,), jnp.float32),
        pltpu.VMEM((L_CH,), jnp.float32),
    ],
)


def _init_xg_body(gx, w, b, ge, out):
  acc = lax.dot_general(gx[...], w[...], (((1,), (1,)), ((), ())),
                        preferred_element_type=jnp.float32)
  out[...] = acc + b[...] + ge[...]


def _init_xg(gx, w, b2, ge):
  blk = 1000
  return pl.pallas_call(
      _init_xg_body,
      grid=(N_G // blk,),
      in_specs=[
          pl.BlockSpec((blk, 1000), lambda i: (i, 0)),
          pl.BlockSpec((D, 1000), lambda i: (0, 0)),
          pl.BlockSpec((1, D), lambda i: (0, 0)),
          pl.BlockSpec((blk, D), lambda i: (i, 0)),
      ],
      out_specs=pl.BlockSpec((blk, D), lambda i: (i, 0)),
      out_shape=jax.ShapeDtypeStruct((N_G, D), jnp.float32),
  )(gx, w, b2, ge)


def _make_transform_body(planes, relu):
  def body(x, agg, cnt, wl, wr, b, out):
    a = agg[0]
    n = cnt[0, :, 0:1]
    for p in range(1, planes):
      a = a + agg[p]
      n = n + cnt[p, :, 0:1]
    mean = a / jnp.maximum(n, 1.0)
    o = (lax.dot_general(mean, wl[...], (((1,), (1,)), ((), ())),
                         preferred_element_type=jnp.float32)
         + lax.dot_general(x[...], wr[...], (((1,), (1,)), ((), ())),
                           preferred_element_type=jnp.float32)
         + b[...])
    if relu:
      o = jnp.maximum(o, 0.0)
    out[...] = o
  return body


def _transform(x, agg, cnt, wl, wr, b2, relu, blk):
  planes = agg.shape[0]
  rows = x.shape[0]
  return pl.pallas_call(
      _make_transform_body(planes, relu),
      grid=(rows // blk,),
      in_specs=[
          pl.BlockSpec((blk, D), lambda i: (i, 0)),
          pl.BlockSpec((planes, blk, D), lambda i: (0, i, 0)),
          pl.BlockSpec((planes, blk, 8), lambda i: (0, i, 0)),
          pl.BlockSpec((D, D), lambda i: (0, 0)),
          pl.BlockSpec((D, D), lambda i: (0, 0)),
          pl.BlockSpec((1, D), lambda i: (0, 0)),
      ],
      out_specs=pl.BlockSpec((blk, D), lambda i: (i, 0)),
      out_shape=jax.ShapeDtypeStruct((rows, D), jnp.float32),
  )(x, agg, cnt, wl, wr, b2)


_agg_p = _make_agg("p")
_agg_g = _make_agg("g")
_cnt_p = _make_cnt("p")
_cnt_g = _make_cnt("g")


def kernel(protein_n_id, go_term_n_id, go_term_x, e_gp_src, e_gp_dst,
           e_pg_src, e_pg_dst, label_src, label_dst, protein_emb,
           go_term_emb, lin_W, lin_b, Wl, bl, Wr):
  f32 = jnp.float32
  xp = jnp.concatenate(
      [protein_emb, jnp.zeros((P_PAD - N_P, D), f32)], axis=0)
  xg = _init_xg(go_term_x, lin_W, lin_b.reshape(1, D), go_term_emb)

  epad = E_PAD - E
  zpad_i = jnp.zeros((epad,), jnp.int32)
  npad_i = jnp.full((epad,), -1, jnp.int32)
  gp_s = jnp.concatenate([e_gp_src, zpad_i])
  gp_d = jnp.concatenate([e_gp_dst, npad_i])
  pg_s = jnp.concatenate([e_pg_src, zpad_i])
  pg_d = jnp.concatenate([e_pg_dst, npad_i])

  z64_p = jnp.zeros((_acc_rows(HALF), D), f32)
  z64_g = jnp.zeros((_acc_rows(N_G), D), f32)
  z8_p = jnp.zeros((_acc_rows(HALF), 8), f32)
  z8_g = jnp.zeros((_acc_rows(N_G), 8), f32)
  ones8 = jnp.ones((E_CH, 8), f32)

  cnt_p = _cnt_p(gp_d, z8_p, ones8).reshape(1, P_PAD, 8)
  cnt_g = _cnt_g(pg_d, z8_g, ones8)

  for layer in range(3):
    relu = layer < 2
    agg_p = _agg_p(xg, gp_s, gp_d, z64_p).reshape(1, P_PAD, D)
    agg_g = _agg_g(xp, pg_s, pg_d, z64_g)
    xp = _transform(xp, agg_p, cnt_p, Wl[2 * layer], Wr[2 * layer],
                    bl[2 * layer].reshape(1, D), relu, 512)
    xg = _transform(xg, agg_g, cnt_g, Wl[2 * layer + 1], Wr[2 * layer + 1],
                    bl[2 * layer + 1].reshape(1, D), relu, 1000)

  lpad = L_PAD - L
  ls = jnp.concatenate([label_src, jnp.zeros((lpad,), jnp.int32)])
  ld = jnp.concatenate([label_dst, jnp.zeros((lpad,), jnp.int32)])
  pred = _classifier(xp, xg, ls, ld)
  return pred[:L]


# speedup vs baseline: 13.7564x; 1.4816x over previous
import functools

import jax
import jax.numpy as jnp
from jax import lax
from jax.experimental import pallas as pl
from jax.experimental.pallas import tpu as pltpu
from jax.experimental.pallas import tpu_sc as plsc

N_P, N_G, D, E, L = 50000, 10000, 64, 800000, 100000

NC, NS = 2, 16
HALF = 25088
P_PAD = 2 * HALF
E_CH = 512
E_PAD = 802816
L_TILE = 3136
L_CH = 448
L_PAD = 32 * L_TILE
EPT = E_PAD // 32
P_CH = 224
EPT_OUT = EPT + 2 * P_CH

_MESH = plsc.VectorSubcoreMesh(
    core_axis_name="c", subcore_axis_name="s", num_cores=NC, num_subcores=NS)
_SC_PARAMS = pltpu.CompilerParams(
    use_tc_tiling_on_sc=False, needs_layout_passes=False)


def _iota16():
  return lax.iota(jnp.int32, 16)


def _acc_rows(own):
  return -(-(own + 16) // 128) * 128


def _out_rows(own):
  r = own // 16
  return own if r % 8 == 0 else _acc_rows(own)


def _make_agg(mode):
  if mode == "p":
    own, edges_per_tile, ch = HALF, E_PAD // NS, 224
  else:
    own, edges_per_tile, ch = N_G, E_PAD // (NC * NS), 448
  acc_rows = _acc_rows(own)
  out_rows = _out_rows(own)
  n_chunks = edges_per_tile // ch
  assert n_chunks % 2 == 0
  zrows = acc_rows // 16
  orows = out_rows // 16

  def body(table, srcp, dstp, zinit, out,
           sidx0, sidx1, didx0, didx1, rows0, rows1, sem0, sem1, acc):
    c = lax.axis_index("c")
    s = lax.axis_index("s")
    z0 = pl.multiple_of(s * zrows, zrows)
    pltpu.sync_copy(zinit.at[pl.ds(z0, zrows)], acc.at[pl.ds(z0, zrows)])
    plsc.subcore_barrier()
    if mode == "p":
      ebase = s * edges_per_tile
      row_base = c * HALF
    else:
      ebase = (c * NS + s) * edges_per_tile
      row_base = 0
    tr = own + _iota16()
    sidx = (sidx0, sidx1)
    didx = (didx0, didx1)
    rows = (rows0, rows1)
    sem = (sem0, sem1)

    def load(i, b):
      eb = pl.multiple_of(ebase + i * ch, 32)
      pltpu.sync_copy(srcp.at[pl.ds(eb, ch)], sidx[b])
      pltpu.sync_copy(dstp.at[pl.ds(eb, ch)], didx[b])
      pltpu.async_copy(table.at[sidx[b]], rows[b], sem[b])

    load(0, 0)

    def pair(i2, carry):
      i = i2 * 2
      for b in (0, 1):
        @pl.when(i + b + 1 < n_chunks)
        def _():
          load(i + b + 1, 1 - b)
        for j in range(ch // 16):
          d = didx[b][pl.ds(j * 16, 16)]
          loc = d - row_base
          ok = (loc >= 0) & (loc < own)
          didx[b][pl.ds(j * 16, 16)] = jnp.where(ok, loc, tr)
        pltpu.make_async_copy(table.at[sidx[b]], rows[b], sem[b]).wait()
        pltpu.sync_copy(rows[b], acc.at[didx[b]], add=True)
      return carry

    lax.fori_loop(0, n_chunks // 2, pair, 0)
    plsc.subcore_barrier()
    o0 = pl.multiple_of(s * orows, orows)
    pltpu.sync_copy(acc.at[pl.ds(o0, orows)], out.at[c, pl.ds(o0, orows)])

  return pl.kernel(
      body,
      out_type=jax.ShapeDtypeStruct((NC, out_rows, D), jnp.float32),
      mesh=_MESH,
      compiler_params=_SC_PARAMS,
      scratch_types=[
          pltpu.VMEM((ch,), jnp.int32),
          pltpu.VMEM((ch,), jnp.int32),
          pltpu.VMEM((ch,), jnp.int32),
          pltpu.VMEM((ch,), jnp.int32),
          pltpu.VMEM((ch, D), jnp.float32),
          pltpu.VMEM((ch, D), jnp.float32),
          pltpu.SemaphoreType.DMA,
          pltpu.SemaphoreType.DMA,
          pltpu.VMEM_SHARED((acc_rows, D), jnp.float32),
      ],
  )


def _part_body(srcp, dstp, ps0, pd0, ps1, pd1, cnts,
               sidx, didx, b0s, b0d, b1s, b1d, cv):
  c = lax.axis_index("c")
  s = lax.axis_index("s")
  t = c * NS + s
  ebase = t * EPT
  n_chunks = EPT // E_CH

  def chunk(i, pos):
    pos0, pos1 = pos
    eb = pl.multiple_of(ebase + i * E_CH, 32)
    pltpu.sync_copy(srcp.at[pl.ds(eb, E_CH)], sidx)
    pltpu.sync_copy(dstp.at[pl.ds(eb, E_CH)], didx)
    for j in range(E_CH // 16):
      sv = sidx[pl.ds(j * 16, 16)]
      dv = didx[pl.ds(j * 16, 16)]
      ok0 = (dv >= 0) & (dv < HALF)
      ok1 = dv >= HALF
      cs0 = plsc.cumsum(jnp.where(ok0, 1, 0))
      plsc.store_scatter(b0s, [pos0 + cs0 - 1], sv, mask=ok0)
      plsc.store_scatter(b0d, [pos0 + cs0 - 1], dv, mask=ok0)
      pos0 = pos0 + lax.reduce_max(cs0, (0,))
      cs1 = plsc.cumsum(jnp.where(ok1, 1, 0))
      plsc.store_scatter(b1s, [pos1 + cs1 - 1], sv, mask=ok1)
      plsc.store_scatter(b1d, [pos1 + cs1 - 1], dv - HALF, mask=ok1)
      pos1 = pos1 + lax.reduce_max(cs1, (0,))
    return (pos0, pos1)

  pos0, pos1 = lax.fori_loop(0, n_chunks, chunk,
                             (jnp.int32(0), jnp.int32(0)))
  for bs, bd, pos in ((b0s, b0d, pos0), (b1s, b1d, pos1)):
    for k in range(2 * P_CH // 16):
      io = pos + k * 16 + _iota16()
      plsc.store_scatter(bs, [io], (k % 16) * 16 + _iota16())
      plsc.store_scatter(bd, [io], HALF + _iota16())
  for ci, pos in ((0, pos0), (1, pos1)):
    nch = ((pos + 2 * P_CH - 1) // (2 * P_CH)) * 2
    cv[...] = jnp.broadcast_to(nch, (16,)).astype(jnp.int32)
    pltpu.sync_copy(cv, cnts.at[ci, t])
  o = pl.multiple_of(t * EPT_OUT, 32)
  pltpu.sync_copy(b0s, ps0.at[pl.ds(o, EPT_OUT)])
  pltpu.sync_copy(b0d, pd0.at[pl.ds(o, EPT_OUT)])
  pltpu.sync_copy(b1s, ps1.at[pl.ds(o, EPT_OUT)])
  pltpu.sync_copy(b1d, pd1.at[pl.ds(o, EPT_OUT)])


_partition = pl.kernel(
    _part_body,
    out_type=[jax.ShapeDtypeStruct((32 * EPT_OUT,), jnp.int32)] * 4
    + [jax.ShapeDtypeStruct((NC, 32, 16), jnp.int32)],
    mesh=_MESH,
    compiler_params=_SC_PARAMS,
    scratch_types=[
        pltpu.VMEM((E_CH,), jnp.int32),
        pltpu.VMEM((E_CH,), jnp.int32),
        pltpu.VMEM((EPT_OUT,), jnp.int32),
        pltpu.VMEM((EPT_OUT,), jnp.int32),
        pltpu.VMEM((EPT_OUT,), jnp.int32),
        pltpu.VMEM((EPT_OUT,), jnp.int32),
        pltpu.VMEM((16,), jnp.int32),
    ],
)


def _agg_p_split_body(table, ps0, pd0, ps1, pd1, cnts, zinit, out,
                      sidx0, sidx1, didx0, didx1, rows0, rows1,
                      sem0, sem1, cv, acc):
  acc_rows = _acc_rows(HALF)
  zrows = acc_rows // 16
  orows = HALF // 16
  c = lax.axis_index("c")
  s = lax.axis_index("s")
  z0 = pl.multiple_of(s * zrows, zrows)
  pltpu.sync_copy(zinit.at[pl.ds(z0, zrows)], acc.at[pl.ds(z0, zrows)])
  plsc.subcore_barrier()
  sidx = (sidx0, sidx1)
  didx = (didx0, didx1)
  rows = (rows0, rows1)
  sem = (sem0, sem1)

  def run_region(bs, bd, ci, t):
    pltpu.sync_copy(cnts.at[ci, t], cv)
    nc = lax.reduce_max(cv[...], (0,))
    base = t * EPT_OUT

    def load(i, b):
      eb = pl.multiple_of(base + i * P_CH, 32)
      pltpu.sync_copy(bs.at[pl.ds(eb, P_CH)], sidx[b])
      pltpu.sync_copy(bd.at[pl.ds(eb, P_CH)], didx[b])
      pltpu.async_copy(table.at[sidx[b]], rows[b], sem[b])

    @pl.when(nc > 0)
    def _():
      load(0, 0)

    def pair(i2, carry):
      i = i2 * 2
      for b in (0, 1):
        @pl.when(i + b + 1 < nc)
        def _():
          load(i + b + 1, 1 - b)
        pltpu.make_async_copy(table.at[sidx[b]], rows[b], sem[b]).wait()
        pltpu.sync_copy(rows[b], acc.at[didx[b]], add=True)
      return carry

    lax.fori_loop(0, nc // 2, pair, 0)

  for r in (0, 1):
    t = s + r * NS
    @pl.when(c == 0)
    def _():
      run_region(ps0, pd0, 0, t)
    @pl.when(c == 1)
    def _():
      run_region(ps1, pd1, 1, t)

  plsc.subcore_barrier()
  o0 = pl.multiple_of(s * orows, orows)
  pltpu.sync_copy(acc.at[pl.ds(o0, orows)], out.at[c, pl.ds(o0, orows)])


_agg_p_split = pl.kernel(
    _agg_p_split_body,
    out_type=jax.ShapeDtypeStruct((NC, HALF, D), jnp.float32),
    mesh=_MESH,
    compiler_params=_SC_PARAMS,
    scratch_types=[
        pltpu.VMEM((P_CH,), jnp.int32),
        pltpu.VMEM((P_CH,), jnp.int32),
        pltpu.VMEM((P_CH,), jnp.int32),
        pltpu.VMEM((P_CH,), jnp.int32),
        pltpu.VMEM((P_CH, D), jnp.float32),
        pltpu.VMEM((P_CH, D), jnp.float32),
        pltpu.SemaphoreType.DMA,
        pltpu.SemaphoreType.DMA,
        pltpu.VMEM((16,), jnp.int32),
        pltpu.VMEM_SHARED((_acc_rows(HALF), D), jnp.float32),
    ],
)


def _make_cnt(mode):
  if mode == "p":
    own, edges_per_tile = HALF, E_PAD // NS
  else:
    own, edges_per_tile = N_G, E_PAD // (NC * NS)
  acc_rows = _acc_rows(own)
  out_rows = _out_rows(own)
  n_chunks = edges_per_tile // E_CH
  zrows = acc_rows // 16
  orows = out_rows // 16

  def body(dstp, zinit, ones, out, didx, dloc, ones_v, acc):
    c = lax.axis_index("c")
    s = lax.axis_index("s")
    z0 = pl.multiple_of(s * zrows, zrows)
    pltpu.sync_copy(zinit.at[pl.ds(z0, zrows)], acc.at[pl.ds(z0, zrows)])
    pltpu.sync_copy(ones, ones_v)
    plsc.subcore_barrier()
    if mode == "p":
      ebase = s * edges_per_tile
      row_base = c * HALF
    else:
      ebase = (c * NS + s) * edges_per_tile
      row_base = 0
    tr = own + _iota16()

    def chunk(i, carry):
      eb = pl.multiple_of(ebase + i * E_CH, E_CH)
      pltpu.sync_copy(dstp.at[pl.ds(eb, E_CH)], didx)
      for j in range(E_CH // 16):
        d = didx[pl.ds(j * 16, 16)]
        loc = d - row_base
        ok = (loc >= 0) & (loc < own)
        dloc[pl.ds(j * 16, 16)] = jnp.where(ok, loc, tr)
      pltpu.sync_copy(ones_v, acc.at[dloc], add=True)
      return carry

    lax.fori_loop(0, n_chunks, chunk, 0)
    plsc.subcore_barrier()
    o0 = pl.multiple_of(s * orows, orows)
    pltpu.sync_copy(acc.at[pl.ds(o0, orows)], out.at[c, pl.ds(o0, orows)])

  return pl.kernel(
      body,
      out_type=jax.ShapeDtypeStruct((NC, out_rows, 8), jnp.float32),
      mesh=_MESH,
      compiler_params=_SC_PARAMS,
      scratch_types=[
          pltpu.VMEM((E_CH,), jnp.int32),
          pltpu.VMEM((E_CH,), jnp.int32),
          pltpu.VMEM((E_CH, 8), jnp.float32),
          pltpu.VMEM_SHARED((acc_rows, 8), jnp.float32),
      ],
  )


def _cls_body(xp, xg, ls, ld, out,
              sidx0, sidx1, didx0, didx1, rp0, rp1, rg0, rg1,
              semp0, semp1, semg0, semg1, tb, ov):
  c = lax.axis_index("c")
  s = lax.axis_index("s")
  base = (c * NS + s) * L_TILE
  ridx = _iota16() * 16
  n_chunks = L_TILE // L_CH
  sidx = (sidx0, sidx1)
  didx = (didx0, didx1)
  rp = (rp0, rp1)
  rg = (rg0, rg1)
  semp = (semp0, semp1)
  semg = (semg0, semg1)

  def load(k, b):
    cb = pl.multiple_of(base + k * L_CH, 32)
    pltpu.sync_copy(ls.at[pl.ds(cb, L_CH)], sidx[b])
    pltpu.sync_copy(ld.at[pl.ds(cb, L_CH)], didx[b])
    pltpu.async_copy(xp.at[sidx[b]], rp[b], semp[b])
    pltpu.async_copy(xg.at[didx[b]], rg[b], semg[b])

  load(0, 0)
  for k in range(n_chunks):
    b = k % 2
    if k + 1 < n_chunks:
      load(k + 1, 1 - b)
    pltpu.make_async_copy(xp.at[sidx[b]], rp[b], semp[b]).wait()
    pltpu.make_async_copy(xg.at[didx[b]], rg[b], semg[b]).wait()
    cb = pl.multiple_of(base + k * L_CH, 32)

    def g16(g, carry):
      for j in range(16):
        r = g * 16 + j
        acc = None
        for m in range(4):
          a = rp[b][r, pl.ds(16 * m, 16)]
          v = rg[b][r, pl.ds(16 * m, 16)]
          av = a * v
          acc = av if acc is None else acc + av
        tb[pl.ds(j * 16, 16)] = acc
      tot = jnp.zeros((16,), jnp.float32)
      for m in range(16):
        tot = tot + plsc.load_gather(tb, [ridx + m])
      ov[pl.ds(g * 16, 16)] = tot
      return carry

    lax.fori_loop(0, L_CH // 16, g16, 0)
    pltpu.sync_copy(ov, out.at[pl.ds(cb, L_CH)])


_classifier = pl.kernel(
    _cls_body,
    out_type=jax.ShapeDtypeStruct((L_PAD,), jnp.float32),
    mesh=_MESH,
    compiler_params=_SC_PARAMS,
    scratch_types=[
        pltpu.VMEM((L_CH,), jnp.int32),
        pltpu.VMEM((L_CH,), jnp.int32),
        pltpu.VMEM((L_CH,), jnp.int32),
        pltpu.VMEM((L_CH,), jnp.int32),
        pltpu.VMEM((L_CH, D), jnp.float32),
        pltpu.VMEM((L_CH, D), jnp.float32),
        pltpu.VMEM((L_CH, D), jnp.float32),
        pltpu.VMEM((L_CH, D), jnp.float32),
        pltpu.SemaphoreType.DMA,
        pltpu.SemaphoreType.DMA,
        pltpu.SemaphoreType.DMA,
        pltpu.SemaphoreType.DMA,
        pltpu.VMEM((256,), jnp.float32),
        pltpu.VMEM((L_CH,), jnp.float32),
    ],
)


def _init_xg_body(gx, w, b, ge, out):
  acc = lax.dot_general(gx[...], w[...], (((1,), (1,)), ((), ())),
                        preferred_element_type=jnp.float32)
  out[...] = acc + b[...] + ge[...]


def _init_xg(gx, w, b2, ge):
  blk = 1000
  return pl.pallas_call(
      _init_xg_body,
      grid=(N_G // blk,),
      in_specs=[
          pl.BlockSpec((blk, 1000), lambda i: (i, 0)),
          pl.BlockSpec((D, 1000), lambda i: (0, 0)),
          pl.BlockSpec((1, D), lambda i: (0, 0)),
          pl.BlockSpec((blk, D), lambda i: (i, 0)),
      ],
      out_specs=pl.BlockSpec((blk, D), lambda i: (i, 0)),
      out_shape=jax.ShapeDtypeStruct((N_G, D), jnp.float32),
  )(gx, w, b2, ge)


def _make_transform_body(planes, relu):
  def body(x, agg, cnt, wl, wr, b, out):
    a = agg[0]
    n = cnt[0, :, 0:1]
    for p in range(1, planes):
      a = a + agg[p]
      n = n + cnt[p, :, 0:1]
    mean = a / jnp.maximum(n, 1.0)
    o = (lax.dot_general(mean, wl[...], (((1,), (1,)), ((), ())),
                         preferred_element_type=jnp.float32)
         + lax.dot_general(x[...], wr[...], (((1,), (1,)), ((), ())),
                           preferred_element_type=jnp.float32)
         + b[...])
    if relu:
      o = jnp.maximum(o, 0.0)
    out[...] = o
  return body


def _transform(x, agg, cnt, wl, wr, b2, relu, blk):
  planes = agg.shape[0]
  rows = x.shape[0]
  return pl.pallas_call(
      _make_transform_body(planes, relu),
      grid=(rows // blk,),
      in_specs=[
          pl.BlockSpec((blk, D), lambda i: (i, 0)),
          pl.BlockSpec((planes, blk, D), lambda i: (0, i, 0)),
          pl.BlockSpec((planes, blk, 8), lambda i: (0, i, 0)),
          pl.BlockSpec((D, D), lambda i: (0, 0)),
          pl.BlockSpec((D, D), lambda i: (0, 0)),
          pl.BlockSpec((1, D), lambda i: (0, 0)),
      ],
      out_specs=pl.BlockSpec((blk, D), lambda i: (i, 0)),
      out_shape=jax.ShapeDtypeStruct((rows, D), jnp.float32),
  )(x, agg, cnt, wl, wr, b2)


_agg_g = _make_agg("g")
_cnt_p = _make_cnt("p")
_cnt_g = _make_cnt("g")


def kernel(protein_n_id, go_term_n_id, go_term_x, e_gp_src, e_gp_dst,
           e_pg_src, e_pg_dst, label_src, label_dst, protein_emb,
           go_term_emb, lin_W, lin_b, Wl, bl, Wr):
  f32 = jnp.float32
  xp = jnp.concatenate(
      [protein_emb, jnp.zeros((P_PAD - N_P, D), f32)], axis=0)
  xg = _init_xg(go_term_x, lin_W, lin_b.reshape(1, D), go_term_emb)

  epad = E_PAD - E
  zpad_i = jnp.zeros((epad,), jnp.int32)
  npad_i = jnp.full((epad,), -1, jnp.int32)
  gp_s = jnp.concatenate([e_gp_src, zpad_i])
  gp_d = jnp.concatenate([e_gp_dst, npad_i])
  pg_s = jnp.concatenate([e_pg_src, zpad_i])
  pg_d = jnp.concatenate([e_pg_dst, npad_i])

  z64_p = jnp.zeros((_acc_rows(HALF), D), f32)
  z64_g = jnp.zeros((_acc_rows(N_G), D), f32)
  z8_p = jnp.zeros((_acc_rows(HALF), 8), f32)
  z8_g = jnp.zeros((_acc_rows(N_G), 8), f32)
  ones8 = jnp.ones((E_CH, 8), f32)

  ps0, pd0, ps1, pd1, pcnts = _partition(gp_s, gp_d)
  cnt_p = _cnt_p(gp_d, z8_p, ones8).reshape(1, P_PAD, 8)
  cnt_g = _cnt_g(pg_d, z8_g, ones8)

  for layer in range(3):
    relu = layer < 2
    agg_p = _agg_p_split(xg, ps0, pd0, ps1, pd1, pcnts,
                         z64_p).reshape(1, P_PAD, D)
    agg_g = _agg_g(xp, pg_s, pg_d, z64_g)
    xp = _transform(xp, agg_p, cnt_p, Wl[2 * layer], Wr[2 * layer],
                    bl[2 * layer].reshape(1, D), relu, 512)
    xg = _transform(xg, agg_g, cnt_g, Wl[2 * layer + 1], Wr[2 * layer + 1],
                    bl[2 * layer + 1].reshape(1, D), relu, 1000)

  lpad = L_PAD - L
  ls = jnp.concatenate([label_src, jnp.zeros((lpad,), jnp.int32)])
  ld = jnp.concatenate([label_dst, jnp.zeros((lpad,), jnp.int32)])
  pred = _classifier(xp, xg, ls, ld)
  return pred[:L]


# speedup vs baseline: 13.7929x; 1.0026x over previous
import functools

import jax
import jax.numpy as jnp
from jax import lax
from jax.experimental import pallas as pl
from jax.experimental.pallas import tpu as pltpu
from jax.experimental.pallas import tpu_sc as plsc

N_P, N_G, D, E, L = 50000, 10000, 64, 800000, 100000

NC, NS = 2, 16
HALF = 25088
P_PAD = 2 * HALF
E_CH = 512
E_PAD = 802816
L_TILE = 3136
L_CH = 448
L_PAD = 32 * L_TILE
EPT = E_PAD // 32
P_CH = 224
EPT_OUT = EPT + 2 * P_CH

_MESH = plsc.VectorSubcoreMesh(
    core_axis_name="c", subcore_axis_name="s", num_cores=NC, num_subcores=NS)
_SC_PARAMS = pltpu.CompilerParams(
    use_tc_tiling_on_sc=False, needs_layout_passes=False)


def _iota16():
  return lax.iota(jnp.int32, 16)


def _acc_rows(own):
  return -(-(own + 16) // 128) * 128


def _out_rows(own):
  r = own // 16
  return own if r % 8 == 0 else _acc_rows(own)


def _make_agg_g():
  own, edges_per_tile, ch = N_G, E_PAD // (NC * NS), 448
  acc_rows = _acc_rows(own)
  out_rows = _out_rows(own)
  n_chunks = edges_per_tile // ch
  assert n_chunks % 2 == 0
  zrows = acc_rows // 16
  orows = out_rows // 16

  def body(table, srcp, dstp, zinit, out,
           sidx0, sidx1, didx0, didx1, rows0, rows1, sem0, sem1, acc):
    c = lax.axis_index("c")
    s = lax.axis_index("s")
    z0 = pl.multiple_of(s * zrows, zrows)
    pltpu.sync_copy(zinit.at[pl.ds(z0, zrows)], acc.at[pl.ds(z0, zrows)])
    plsc.subcore_barrier()
    ebase = (c * NS + s) * edges_per_tile
    sidx = (sidx0, sidx1)
    didx = (didx0, didx1)
    rows = (rows0, rows1)
    sem = (sem0, sem1)

    def load(i, b):
      eb = pl.multiple_of(ebase + i * ch, 32)
      pltpu.sync_copy(srcp.at[pl.ds(eb, ch)], sidx[b])
      pltpu.sync_copy(dstp.at[pl.ds(eb, ch)], didx[b])
      pltpu.async_copy(table.at[sidx[b]], rows[b], sem[b])

    load(0, 0)

    def pair(i2, carry):
      i = i2 * 2
      for b in (0, 1):
        @pl.when(i + b + 1 < n_chunks)
        def _():
          load(i + b + 1, 1 - b)
        pltpu.make_async_copy(table.at[sidx[b]], rows[b], sem[b]).wait()
        pltpu.sync_copy(rows[b], acc.at[didx[b]], add=True)
      return carry

    lax.fori_loop(0, n_chunks // 2, pair, 0)
    plsc.subcore_barrier()
    o0 = pl.multiple_of(s * orows, orows)
    pltpu.sync_copy(acc.at[pl.ds(o0, orows)], out.at[c, pl.ds(o0, orows)])

  return pl.kernel(
      body,
      out_type=jax.ShapeDtypeStruct((NC, out_rows, D), jnp.float32),
      mesh=_MESH,
      compiler_params=_SC_PARAMS,
      scratch_types=[
          pltpu.VMEM((ch,), jnp.int32),
          pltpu.VMEM((ch,), jnp.int32),
          pltpu.VMEM((ch,), jnp.int32),
          pltpu.VMEM((ch,), jnp.int32),
          pltpu.VMEM((ch, D), jnp.float32),
          pltpu.VMEM((ch, D), jnp.float32),
          pltpu.SemaphoreType.DMA,
          pltpu.SemaphoreType.DMA,
          pltpu.VMEM_SHARED((acc_rows, D), jnp.float32),
      ],
  )


def _part_body(srcp, dstp, ps0, pd0, ps1, pd1, cnts,
               sidx, didx, b0s, b0d, b1s, b1d, cv):
  c = lax.axis_index("c")
  s = lax.axis_index("s")
  t = c * NS + s
  ebase = t * EPT
  n_chunks = EPT // E_CH

  def chunk(i, pos):
    pos0, pos1 = pos
    eb = pl.multiple_of(ebase + i * E_CH, 32)
    pltpu.sync_copy(srcp.at[pl.ds(eb, E_CH)], sidx)
    pltpu.sync_copy(dstp.at[pl.ds(eb, E_CH)], didx)
    for j in range(E_CH // 16):
      sv = sidx[pl.ds(j * 16, 16)]
      dv = didx[pl.ds(j * 16, 16)]
      ok0 = (dv >= 0) & (dv < HALF)
      ok1 = dv >= HALF
      cs0 = plsc.cumsum(jnp.where(ok0, 1, 0))
      plsc.store_scatter(b0s, [pos0 + cs0 - 1], sv, mask=ok0)
      plsc.store_scatter(b0d, [pos0 + cs0 - 1], dv, mask=ok0)
      pos0 = pos0 + lax.reduce_max(cs0, (0,))
      cs1 = plsc.cumsum(jnp.where(ok1, 1, 0))
      plsc.store_scatter(b1s, [pos1 + cs1 - 1], sv, mask=ok1)
      plsc.store_scatter(b1d, [pos1 + cs1 - 1], dv - HALF, mask=ok1)
      pos1 = pos1 + lax.reduce_max(cs1, (0,))
    return (pos0, pos1)

  pos0, pos1 = lax.fori_loop(0, n_chunks, chunk,
                             (jnp.int32(0), jnp.int32(0)))
  for bs, bd, pos in ((b0s, b0d, pos0), (b1s, b1d, pos1)):
    for k in range(2 * P_CH // 16):
      io = pos + k * 16 + _iota16()
      plsc.store_scatter(bs, [io], (k % 16) * 16 + _iota16())
      plsc.store_scatter(bd, [io], HALF + _iota16())
  for ci, pos in ((0, pos0), (1, pos1)):
    nch = ((pos + 2 * P_CH - 1) // (2 * P_CH)) * 2
    cv[...] = jnp.broadcast_to(nch, (16,)).astype(jnp.int32)
    pltpu.sync_copy(cv, cnts.at[ci, t])
  o = pl.multiple_of(t * EPT_OUT, 32)
  pltpu.sync_copy(b0s, ps0.at[pl.ds(o, EPT_OUT)])
  pltpu.sync_copy(b0d, pd0.at[pl.ds(o, EPT_OUT)])
  pltpu.sync_copy(b1s, ps1.at[pl.ds(o, EPT_OUT)])
  pltpu.sync_copy(b1d, pd1.at[pl.ds(o, EPT_OUT)])


_partition = pl.kernel(
    _part_body,
    out_type=[jax.ShapeDtypeStruct((32 * EPT_OUT,), jnp.int32)] * 4
    + [jax.ShapeDtypeStruct((NC, 32, 16), jnp.int32)],
    mesh=_MESH,
    compiler_params=_SC_PARAMS,
    scratch_types=[
        pltpu.VMEM((E_CH,), jnp.int32),
        pltpu.VMEM((E_CH,), jnp.int32),
        pltpu.VMEM((EPT_OUT,), jnp.int32),
        pltpu.VMEM((EPT_OUT,), jnp.int32),
        pltpu.VMEM((EPT_OUT,), jnp.int32),
        pltpu.VMEM((EPT_OUT,), jnp.int32),
        pltpu.VMEM((16,), jnp.int32),
    ],
)


def _agg_p_split_body(table, ps0, pd0, ps1, pd1, cnts, zinit, out,
                      sidx0, sidx1, didx0, didx1, rows0, rows1,
                      sem0, sem1, cv, acc):
  acc_rows = _acc_rows(HALF)
  zrows = acc_rows // 16
  orows = HALF // 16
  c = lax.axis_index("c")
  s = lax.axis_index("s")
  z0 = pl.multiple_of(s * zrows, zrows)
  pltpu.sync_copy(zinit.at[pl.ds(z0, zrows)], acc.at[pl.ds(z0, zrows)])
  plsc.subcore_barrier()
  sidx = (sidx0, sidx1)
  didx = (didx0, didx1)
  rows = (rows0, rows1)
  sem = (sem0, sem1)

  def run_region(bs, bd, ci, t):
    pltpu.sync_copy(cnts.at[ci, t], cv)
    nc = lax.reduce_max(cv[...], (0,))
    base = t * EPT_OUT

    def load(i, b):
      eb = pl.multiple_of(base + i * P_CH, 32)
      pltpu.sync_copy(bs.at[pl.ds(eb, P_CH)], sidx[b])
      pltpu.sync_copy(bd.at[pl.ds(eb, P_CH)], didx[b])
      pltpu.async_copy(table.at[sidx[b]], rows[b], sem[b])

    @pl.when(nc > 0)
    def _():
      load(0, 0)

    def pair(i2, carry):
      i = i2 * 2
      for b in (0, 1):
        @pl.when(i + b + 1 < nc)
        def _():
          load(i + b + 1, 1 - b)
        pltpu.make_async_copy(table.at[sidx[b]], rows[b], sem[b]).wait()
        pltpu.sync_copy(rows[b], acc.at[didx[b]], add=True)
      return carry

    lax.fori_loop(0, nc // 2, pair, 0)

  for r in (0, 1):
    t = s + r * NS
    @pl.when(c == 0)
    def _():
      run_region(ps0, pd0, 0, t)
    @pl.when(c == 1)
    def _():
      run_region(ps1, pd1, 1, t)

  plsc.subcore_barrier()
  o0 = pl.multiple_of(s * orows, orows)
  pltpu.sync_copy(acc.at[pl.ds(o0, orows)], out.at[c, pl.ds(o0, orows)])


_agg_p_split = pl.kernel(
    _agg_p_split_body,
    out_type=jax.ShapeDtypeStruct((NC, HALF, D), jnp.float32),
    mesh=_MESH,
    compiler_params=_SC_PARAMS,
    scratch_types=[
        pltpu.VMEM((P_CH,), jnp.int32),
        pltpu.VMEM((P_CH,), jnp.int32),
        pltpu.VMEM((P_CH,), jnp.int32),
        pltpu.VMEM((P_CH,), jnp.int32),
        pltpu.VMEM((P_CH, D), jnp.float32),
        pltpu.VMEM((P_CH, D), jnp.float32),
        pltpu.SemaphoreType.DMA,
        pltpu.SemaphoreType.DMA,
        pltpu.VMEM((16,), jnp.int32),
        pltpu.VMEM_SHARED((_acc_rows(HALF), D), jnp.float32),
    ],
)


def _cnts_body(pd0, pd1, pgd, cnts, zp, zg, ones, out_p, out_g,
               didx_p, didx, ones_v, cv, accp, accg):
  accp_rows = _acc_rows(HALF)
  accg_rows = _acc_rows(N_G)
  c = lax.axis_index("c")
  s = lax.axis_index("s")
  zpr = accp_rows // 16
  zgr = accg_rows // 16
  zp0 = pl.multiple_of(s * zpr, zpr)
  pltpu.sync_copy(zp.at[pl.ds(zp0, zpr)], accp.at[pl.ds(zp0, zpr)])
  zg0 = pl.multiple_of(s * zgr, zgr)
  pltpu.sync_copy(zg.at[pl.ds(zg0, zgr)], accg.at[pl.ds(zg0, zgr)])
  pltpu.sync_copy(ones, ones_v)
  plsc.subcore_barrier()

  def run_region(bd, ci, t):
    pltpu.sync_copy(cnts.at[ci, t], cv)
    nc = lax.reduce_max(cv[...], (0,))
    base = t * EPT_OUT

    def chunk(i, carry):
      eb = pl.multiple_of(base + i * P_CH, 32)
      pltpu.sync_copy(bd.at[pl.ds(eb, P_CH)], didx_p)
      pltpu.sync_copy(ones_v.at[pl.ds(0, P_CH)], accp.at[didx_p], add=True)
      return carry

    lax.fori_loop(0, nc, chunk, 0)

  for r in (0, 1):
    t = s + r * NS
    @pl.when(c == 0)
    def _():
      run_region(pd0, 0, t)
    @pl.when(c == 1)
    def _():
      run_region(pd1, 1, t)

  ept_g = E_PAD // (NC * NS)
  ebase = (c * NS + s) * ept_g

  def chunk_g(i, carry):
    eb = pl.multiple_of(ebase + i * E_CH, E_CH)
    pltpu.sync_copy(pgd.at[pl.ds(eb, E_CH)], didx)
    pltpu.sync_copy(ones_v, accg.at[didx], add=True)
    return carry

  lax.fori_loop(0, ept_g // E_CH, chunk_g, 0)
  plsc.subcore_barrier()
  opr = HALF // 16
  op0 = pl.multiple_of(s * opr, opr)
  pltpu.sync_copy(accp.at[pl.ds(op0, opr)], out_p.at[c, pl.ds(op0, opr)])
  ogr = _out_rows(N_G) // 16
  og0 = pl.multiple_of(s * ogr, ogr)
  pltpu.sync_copy(accg.at[pl.ds(og0, ogr)], out_g.at[c, pl.ds(og0, ogr)])


_counts_all = pl.kernel(
    _cnts_body,
    out_type=[jax.ShapeDtypeStruct((NC, HALF, 8), jnp.float32),
              jax.ShapeDtypeStruct((NC, _out_rows(N_G), 8), jnp.float32)],
    mesh=_MESH,
    compiler_params=_SC_PARAMS,
    scratch_types=[
        pltpu.VMEM((P_CH,), jnp.int32),
        pltpu.VMEM((E_CH,), jnp.int32),
        pltpu.VMEM((E_CH, 8), jnp.float32),
        pltpu.VMEM((16,), jnp.int32),
        pltpu.VMEM_SHARED((_acc_rows(HALF), 8), jnp.float32),
        pltpu.VMEM_SHARED((_acc_rows(N_G), 8), jnp.float32),
    ],
)


def _cls_body(xp, xg, ls, ld, out,
              sidx0, sidx1, didx0, didx1, rp0, rp1, rg0, rg1,
              semp0, semp1, semg0, semg1, tb, ov):
  c = lax.axis_index("c")
  s = lax.axis_index("s")
  base = (c * NS + s) * L_TILE
  ridx = _iota16() * 16
  n_chunks = L_TILE // L_CH
  sidx = (sidx0, sidx1)
  didx = (didx0, didx1)
  rp = (rp0, rp1)
  rg = (rg0, rg1)
  semp = (semp0, semp1)
  semg = (semg0, semg1)

  def load(k, b):
    cb = pl.multiple_of(base + k * L_CH, 32)
    pltpu.sync_copy(ls.at[pl.ds(cb, L_CH)], sidx[b])
    pltpu.sync_copy(ld.at[pl.ds(cb, L_CH)], didx[b])
    pltpu.async_copy(xp.at[sidx[b]], rp[b], semp[b])
    pltpu.async_copy(xg.at[didx[b]], rg[b], semg[b])

  load(0, 0)
  for k in range(n_chunks):
    b = k % 2
    if k + 1 < n_chunks:
      load(k + 1, 1 - b)
    pltpu.make_async_copy(xp.at[sidx[b]], rp[b], semp[b]).wait()
    pltpu.make_async_copy(xg.at[didx[b]], rg[b], semg[b]).wait()
    cb = pl.multiple_of(base + k * L_CH, 32)

    def g16(g, carry):
      for j in range(16):
        r = g * 16 + j
        acc = None
        for m in range(4):
          a = rp[b][r, pl.ds(16 * m, 16)]
          v = rg[b][r, pl.ds(16 * m, 16)]
          av = a * v
          acc = av if acc is None else acc + av
        tb[pl.ds(j * 16, 16)] = acc
      tot = jnp.zeros((16,), jnp.float32)
      for m in range(16):
        tot = tot + plsc.load_gather(tb, [ridx + m])
      ov[pl.ds(g * 16, 16)] = tot
      return carry

    lax.fori_loop(0, L_CH // 16, g16, 0)
    pltpu.sync_copy(ov, out.at[pl.ds(cb, L_CH)])


_classifier = pl.kernel(
    _cls_body,
    out_type=jax.ShapeDtypeStruct((L_PAD,), jnp.float32),
    mesh=_MESH,
    compiler_params=_SC_PARAMS,
    scratch_types=[
        pltpu.VMEM((L_CH,), jnp.int32),
        pltpu.VMEM((L_CH,), jnp.int32),
        pltpu.VMEM((L_CH,), jnp.int32),
        pltpu.VMEM((L_CH,), jnp.int32),
        pltpu.VMEM((L_CH, D), jnp.float32),
        pltpu.VMEM((L_CH, D), jnp.float32),
        pltpu.VMEM((L_CH, D), jnp.float32),
        pltpu.VMEM((L_CH, D), jnp.float32),
        pltpu.SemaphoreType.DMA,
        pltpu.SemaphoreType.DMA,
        pltpu.SemaphoreType.DMA,
        pltpu.SemaphoreType.DMA,
        pltpu.VMEM((256,), jnp.float32),
        pltpu.VMEM((L_CH,), jnp.float32),
    ],
)


def _init_xg_body(gx, w, b, ge, out):
  acc = lax.dot_general(gx[...], w[...], (((1,), (1,)), ((), ())),
                        preferred_element_type=jnp.float32)
  out[...] = acc + b[...] + ge[...]


def _init_xg(gx, w, b2, ge):
  blk = 1000
  return pl.pallas_call(
      _init_xg_body,
      grid=(N_G // blk,),
      in_specs=[
          pl.BlockSpec((blk, 1000), lambda i: (i, 0)),
          pl.BlockSpec((D, 1000), lambda i: (0, 0)),
          pl.BlockSpec((1, D), lambda i: (0, 0)),
          pl.BlockSpec((blk, D), lambda i: (i, 0)),
      ],
      out_specs=pl.BlockSpec((blk, D), lambda i: (i, 0)),
      out_shape=jax.ShapeDtypeStruct((N_G, D), jnp.float32),
  )(gx, w, b2, ge)


def _make_transform_body(planes, relu):
  def body(x, agg, cnt, wl, wr, b, out):
    a = agg[0]
    n = cnt[0, :, 0:1]
    for p in range(1, planes):
      a = a + agg[p]
      n = n + cnt[p, :, 0:1]
    mean = a / jnp.maximum(n, 1.0)
    o = (lax.dot_general(mean, wl[...], (((1,), (1,)), ((), ())),
                         preferred_element_type=jnp.float32)
         + lax.dot_general(x[...], wr[...], (((1,), (1,)), ((), ())),
                           preferred_element_type=jnp.float32)
         + b[...])
    if relu:
      o = jnp.maximum(o, 0.0)
    out[...] = o
  return body


def _transform(x, agg, cnt, wl, wr, b2, relu, blk):
  planes = agg.shape[0]
  rows = x.shape[0]
  return pl.pallas_call(
      _make_transform_body(planes, relu),
      grid=(rows // blk,),
      in_specs=[
          pl.BlockSpec((blk, D), lambda i: (i, 0)),
          pl.BlockSpec((planes, blk, D), lambda i: (0, i, 0)),
          pl.BlockSpec((planes, blk, 8), lambda i: (0, i, 0)),
          pl.BlockSpec((D, D), lambda i: (0, 0)),
          pl.BlockSpec((D, D), lambda i: (0, 0)),
          pl.BlockSpec((1, D), lambda i: (0, 0)),
      ],
      out_specs=pl.BlockSpec((blk, D), lambda i: (i, 0)),
      out_shape=jax.ShapeDtypeStruct((rows, D), jnp.float32),
  )(x, agg, cnt, wl, wr, b2)


_agg_g = _make_agg_g()


def kernel(protein_n_id, go_term_n_id, go_term_x, e_gp_src, e_gp_dst,
           e_pg_src, e_pg_dst, label_src, label_dst, protein_emb,
           go_term_emb, lin_W, lin_b, Wl, bl, Wr):
  f32 = jnp.float32
  xp = jnp.concatenate(
      [protein_emb, jnp.zeros((P_PAD - N_P, D), f32)], axis=0)
  xg = _init_xg(go_term_x, lin_W, lin_b.reshape(1, D), go_term_emb)

  epad = E_PAD - E
  zpad_i = jnp.zeros((epad,), jnp.int32)
  npad_i = jnp.full((epad,), -1, jnp.int32)
  gp_s = jnp.concatenate([e_gp_src, zpad_i])
  gp_d = jnp.concatenate([e_gp_dst, npad_i])
  tpad_i = N_G + (jnp.arange(epad, dtype=jnp.int32) % 16)
  pg_s = jnp.concatenate([e_pg_src, zpad_i])
  pg_d = jnp.concatenate([e_pg_dst, tpad_i])

  z64_p = jnp.zeros((_acc_rows(HALF), D), f32)
  z64_g = jnp.zeros((_acc_rows(N_G), D), f32)
  z8_p = jnp.zeros((_acc_rows(HALF), 8), f32)
  z8_g = jnp.zeros((_acc_rows(N_G), 8), f32)
  ones8 = jnp.ones((E_CH, 8), f32)

  ps0, pd0, ps1, pd1, pcnts = _partition(gp_s, gp_d)
  cnt_p, cnt_g = _counts_all(pd0, pd1, pg_d, pcnts, z8_p, z8_g, ones8)
  cnt_p = cnt_p.reshape(1, P_PAD, 8)

  for layer in range(3):
    relu = layer < 2
    agg_p = _agg_p_split(xg, ps0, pd0, ps1, pd1, pcnts,
                         z64_p).reshape(1, P_PAD, D)
    agg_g = _agg_g(xp, pg_s, pg_d, z64_g)
    xp = _transform(xp, agg_p, cnt_p, Wl[2 * layer], Wr[2 * layer],
                    bl[2 * layer].reshape(1, D), relu, 512)
    xg = _transform(xg, agg_g, cnt_g, Wl[2 * layer + 1], Wr[2 * layer + 1],
                    bl[2 * layer + 1].reshape(1, D), relu, 1000)

  lpad = L_PAD - L
  ls = jnp.concatenate([label_src, jnp.zeros((lpad,), jnp.int32)])
  ld = jnp.concatenate([label_dst, jnp.zeros((lpad,), jnp.int32)])
  pred = _classifier(xp, xg, ls, ld)
  return pred[:L]


# speedup vs baseline: 15.3531x; 1.1131x over previous
import functools

import jax
import jax.numpy as jnp
from jax import lax
from jax.experimental import pallas as pl
from jax.experimental.pallas import tpu as pltpu
from jax.experimental.pallas import tpu_sc as plsc

N_P, N_G, D, E, L = 50000, 10000, 64, 800000, 100000

NC, NS = 2, 16
HALF = 25088
P_PAD = 2 * HALF
E_CH = 512
E_PAD = 802816
L_TILE = 3136
L_CH = 448
L_PAD = 32 * L_TILE
EPT = E_PAD // 32
P_CH = 224
EPT_OUT = EPT + 2 * P_CH

_MESH = plsc.VectorSubcoreMesh(
    core_axis_name="c", subcore_axis_name="s", num_cores=NC, num_subcores=NS)
_SC_PARAMS = pltpu.CompilerParams(
    use_tc_tiling_on_sc=False, needs_layout_passes=False)


def _iota16():
  return lax.iota(jnp.int32, 16)


def _acc_rows(own):
  return -(-(own + 16) // 128) * 128


def _out_rows(own):
  r = own // 16
  return own if r % 8 == 0 else _acc_rows(own)


def _make_agg_g():
  own, edges_per_tile, ch = N_G, E_PAD // (NC * NS), 448
  acc_rows = _acc_rows(own)
  out_rows = _out_rows(own)
  n_chunks = edges_per_tile // ch
  assert n_chunks % 2 == 0
  zrows = acc_rows // 16
  orows = out_rows // 16

  def body(table, srcp, dstp, zinit, out,
           sidx0, sidx1, didx0, didx1, rows0, rows1, sem0, sem1,
           ssem0, ssem1, acc):
    c = lax.axis_index("c")
    s = lax.axis_index("s")
    ssem = (ssem0, ssem1)
    z0 = pl.multiple_of(s * zrows, zrows)
    pltpu.sync_copy(zinit.at[pl.ds(z0, zrows)], acc.at[pl.ds(z0, zrows)])
    plsc.subcore_barrier()
    ebase = (c * NS + s) * edges_per_tile
    sidx = (sidx0, sidx1)
    didx = (didx0, didx1)
    rows = (rows0, rows1)
    sem = (sem0, sem1)

    def load(i, b):
      eb = pl.multiple_of(ebase + i * ch, 32)
      pltpu.sync_copy((srcp.at[pl.ds(eb, ch)], dstp.at[pl.ds(eb, ch)]),
                      (sidx[b], didx[b]))
      pltpu.async_copy(table.at[sidx[b]], rows[b], sem[b])

    load(0, 0)

    def pair(i2, carry):
      i = i2 * 2
      for b in (0, 1):
        @pl.when(i + b >= 1)
        def _():
          pltpu.make_async_copy(
              rows[1 - b], acc.at[didx[1 - b]], ssem[1 - b]).wait()
        @pl.when(i + b + 1 < n_chunks)
        def _():
          load(i + b + 1, 1 - b)
        pltpu.make_async_copy(table.at[sidx[b]], rows[b], sem[b]).wait()
        pltpu.make_async_copy(rows[b], acc.at[didx[b]],
                              ssem[b]).start(add=True)
      return carry

    lax.fori_loop(0, n_chunks // 2, pair, 0)
    pltpu.make_async_copy(rows[1], acc.at[didx[1]], ssem[1]).wait()
    plsc.subcore_barrier()
    o0 = pl.multiple_of(s * orows, orows)
    pltpu.sync_copy(acc.at[pl.ds(o0, orows)], out.at[c, pl.ds(o0, orows)])

  return pl.kernel(
      body,
      out_type=jax.ShapeDtypeStruct((NC, out_rows, D), jnp.float32),
      mesh=_MESH,
      compiler_params=_SC_PARAMS,
      scratch_types=[
          pltpu.VMEM((ch,), jnp.int32),
          pltpu.VMEM((ch,), jnp.int32),
          pltpu.VMEM((ch,), jnp.int32),
          pltpu.VMEM((ch,), jnp.int32),
          pltpu.VMEM((ch, D), jnp.float32),
          pltpu.VMEM((ch, D), jnp.float32),
          pltpu.SemaphoreType.DMA,
          pltpu.SemaphoreType.DMA,
          pltpu.SemaphoreType.DMA,
          pltpu.SemaphoreType.DMA,
          pltpu.VMEM_SHARED((acc_rows, D), jnp.float32),
      ],
  )


def _part_body(srcp, dstp, ps0, pd0, ps1, pd1, cnts,
               sidx, didx, b0s, b0d, b1s, b1d, cv):
  c = lax.axis_index("c")
  s = lax.axis_index("s")
  t = c * NS + s
  ebase = t * EPT
  n_chunks = EPT // E_CH

  def chunk(i, pos):
    pos0, pos1 = pos
    eb = pl.multiple_of(ebase + i * E_CH, 32)
    pltpu.sync_copy(srcp.at[pl.ds(eb, E_CH)], sidx)
    pltpu.sync_copy(dstp.at[pl.ds(eb, E_CH)], didx)
    for j in range(E_CH // 16):
      sv = sidx[pl.ds(j * 16, 16)]
      dv = didx[pl.ds(j * 16, 16)]
      ok0 = (dv >= 0) & (dv < HALF)
      ok1 = dv >= HALF
      cs0 = plsc.cumsum(jnp.where(ok0, 1, 0))
      plsc.store_scatter(b0s, [pos0 + cs0 - 1], sv, mask=ok0)
      plsc.store_scatter(b0d, [pos0 + cs0 - 1], dv, mask=ok0)
      pos0 = pos0 + lax.reduce_max(cs0, (0,))
      cs1 = plsc.cumsum(jnp.where(ok1, 1, 0))
      plsc.store_scatter(b1s, [pos1 + cs1 - 1], sv, mask=ok1)
      plsc.store_scatter(b1d, [pos1 + cs1 - 1], dv - HALF, mask=ok1)
      pos1 = pos1 + lax.reduce_max(cs1, (0,))
    return (pos0, pos1)

  pos0, pos1 = lax.fori_loop(0, n_chunks, chunk,
                             (jnp.int32(0), jnp.int32(0)))
  for bs, bd, pos in ((b0s, b0d, pos0), (b1s, b1d, pos1)):
    for k in range(2 * P_CH // 16):
      io = pos + k * 16 + _iota16()
      plsc.store_scatter(bs, [io], (k % 16) * 16 + _iota16())
      plsc.store_scatter(bd, [io], HALF + _iota16())
  for ci, pos in ((0, pos0), (1, pos1)):
    nch = ((pos + 2 * P_CH - 1) // (2 * P_CH)) * 2
    cv[...] = jnp.broadcast_to(nch, (16,)).astype(jnp.int32)
    pltpu.sync_copy(cv, cnts.at[ci, t])
  o = pl.multiple_of(t * EPT_OUT, 32)
  pltpu.sync_copy(b0s, ps0.at[pl.ds(o, EPT_OUT)])
  pltpu.sync_copy(b0d, pd0.at[pl.ds(o, EPT_OUT)])
  pltpu.sync_copy(b1s, ps1.at[pl.ds(o, EPT_OUT)])
  pltpu.sync_copy(b1d, pd1.at[pl.ds(o, EPT_OUT)])


_partition = pl.kernel(
    _part_body,
    out_type=[jax.ShapeDtypeStruct((32 * EPT_OUT,), jnp.int32)] * 4
    + [jax.ShapeDtypeStruct((NC, 32, 16), jnp.int32)],
    mesh=_MESH,
    compiler_params=_SC_PARAMS,
    scratch_types=[
        pltpu.VMEM((E_CH,), jnp.int32),
        pltpu.VMEM((E_CH,), jnp.int32),
        pltpu.VMEM((EPT_OUT,), jnp.int32),
        pltpu.VMEM((EPT_OUT,), jnp.int32),
        pltpu.VMEM((EPT_OUT,), jnp.int32),
        pltpu.VMEM((EPT_OUT,), jnp.int32),
        pltpu.VMEM((16,), jnp.int32),
    ],
)


def _agg_p_split_body(table, ps0, pd0, ps1, pd1, cnts, zinit, out,
                      sidx0, sidx1, didx0, didx1, rows0, rows1,
                      sem0, sem1, ssem0, ssem1, cv, acc):
  acc_rows = _acc_rows(HALF)
  zrows = acc_rows // 16
  orows = HALF // 16
  c = lax.axis_index("c")
  s = lax.axis_index("s")
  z0 = pl.multiple_of(s * zrows, zrows)
  pltpu.sync_copy(zinit.at[pl.ds(z0, zrows)], acc.at[pl.ds(z0, zrows)])
  plsc.subcore_barrier()
  sidx = (sidx0, sidx1)
  didx = (didx0, didx1)
  rows = (rows0, rows1)
  sem = (sem0, sem1)
  ssem = (ssem0, ssem1)

  def run_region(bs, bd, ci, t):
    pltpu.sync_copy(cnts.at[ci, t], cv)
    nc = lax.reduce_max(cv[...], (0,))
    base = t * EPT_OUT

    def load(i, b):
      eb = pl.multiple_of(base + i * P_CH, 32)
      pltpu.sync_copy((bs.at[pl.ds(eb, P_CH)], bd.at[pl.ds(eb, P_CH)]),
                      (sidx[b], didx[b]))
      pltpu.async_copy(table.at[sidx[b]], rows[b], sem[b])

    @pl.when(nc > 0)
    def _():
      load(0, 0)

    def pair(i2, carry):
      i = i2 * 2
      for b in (0, 1):
        @pl.when(i + b >= 1)
        def _():
          pltpu.make_async_copy(
              rows[1 - b], acc.at[didx[1 - b]], ssem[1 - b]).wait()
        @pl.when(i + b + 1 < nc)
        def _():
          load(i + b + 1, 1 - b)
        pltpu.make_async_copy(table.at[sidx[b]], rows[b], sem[b]).wait()
        pltpu.make_async_copy(rows[b], acc.at[didx[b]],
                              ssem[b]).start(add=True)
      return carry

    lax.fori_loop(0, nc // 2, pair, 0)
    @pl.when(nc > 0)
    def _():
      pltpu.make_async_copy(rows[1], acc.at[didx[1]], ssem[1]).wait()

  for r in (0, 1):
    t = s + r * NS
    @pl.when(c == 0)
    def _():
      run_region(ps0, pd0, 0, t)
    @pl.when(c == 1)
    def _():
      run_region(ps1, pd1, 1, t)

  plsc.subcore_barrier()
  o0 = pl.multiple_of(s * orows, orows)
  pltpu.sync_copy(acc.at[pl.ds(o0, orows)], out.at[c, pl.ds(o0, orows)])


_agg_p_split = pl.kernel(
    _agg_p_split_body,
    out_type=jax.ShapeDtypeStruct((NC, HALF, D), jnp.float32),
    mesh=_MESH,
    compiler_params=_SC_PARAMS,
    scratch_types=[
        pltpu.VMEM((P_CH,), jnp.int32),
        pltpu.VMEM((P_CH,), jnp.int32),
        pltpu.VMEM((P_CH,), jnp.int32),
        pltpu.VMEM((P_CH,), jnp.int32),
        pltpu.VMEM((P_CH, D), jnp.float32),
        pltpu.VMEM((P_CH, D), jnp.float32),
        pltpu.SemaphoreType.DMA,
        pltpu.SemaphoreType.DMA,
        pltpu.SemaphoreType.DMA,
        pltpu.SemaphoreType.DMA,
        pltpu.VMEM((16,), jnp.int32),
        pltpu.VMEM_SHARED((_acc_rows(HALF), D), jnp.float32),
    ],
)


def _cnts_body(pd0, pd1, pgd, cnts, zp, zg, ones, out_p, out_g,
               didx_p, didx, ones_v, cv, accp, accg):
  accp_rows = _acc_rows(HALF)
  accg_rows = _acc_rows(N_G)
  c = lax.axis_index("c")
  s = lax.axis_index("s")
  zpr = accp_rows // 16
  zgr = accg_rows // 16
  zp0 = pl.multiple_of(s * zpr, zpr)
  pltpu.sync_copy(zp.at[pl.ds(zp0, zpr)], accp.at[pl.ds(zp0, zpr)])
  zg0 = pl.multiple_of(s * zgr, zgr)
  pltpu.sync_copy(zg.at[pl.ds(zg0, zgr)], accg.at[pl.ds(zg0, zgr)])
  pltpu.sync_copy(ones, ones_v)
  plsc.subcore_barrier()

  def run_region(bd, ci, t):
    pltpu.sync_copy(cnts.at[ci, t], cv)
    nc = lax.reduce_max(cv[...], (0,))
    base = t * EPT_OUT

    def chunk(i, carry):
      eb = pl.multiple_of(base + i * P_CH, 32)
      pltpu.sync_copy(bd.at[pl.ds(eb, P_CH)], didx_p)
      pltpu.sync_copy(ones_v.at[pl.ds(0, P_CH)], accp.at[didx_p], add=True)
      return carry

    lax.fori_loop(0, nc, chunk, 0)

  for r in (0, 1):
    t = s + r * NS
    @pl.when(c == 0)
    def _():
      run_region(pd0, 0, t)
    @pl.when(c == 1)
    def _():
      run_region(pd1, 1, t)

  ept_g = E_PAD // (NC * NS)
  ebase = (c * NS + s) * ept_g

  def chunk_g(i, carry):
    eb = pl.multiple_of(ebase + i * E_CH, E_CH)
    pltpu.sync_copy(pgd.at[pl.ds(eb, E_CH)], didx)
    pltpu.sync_copy(ones_v, accg.at[didx], add=True)
    return carry

  lax.fori_loop(0, ept_g // E_CH, chunk_g, 0)
  plsc.subcore_barrier()
  opr = HALF // 16
  op0 = pl.multiple_of(s * opr, opr)
  pltpu.sync_copy(accp.at[pl.ds(op0, opr)], out_p.at[c, pl.ds(op0, opr)])
  ogr = _out_rows(N_G) // 16
  og0 = pl.multiple_of(s * ogr, ogr)
  pltpu.sync_copy(accg.at[pl.ds(og0, ogr)], out_g.at[c, pl.ds(og0, ogr)])


_counts_all = pl.kernel(
    _cnts_body,
    out_type=[jax.ShapeDtypeStruct((NC, HALF, 8), jnp.float32),
              jax.ShapeDtypeStruct((NC, _out_rows(N_G), 8), jnp.float32)],
    mesh=_MESH,
    compiler_params=_SC_PARAMS,
    scratch_types=[
        pltpu.VMEM((P_CH,), jnp.int32),
        pltpu.VMEM((E_CH,), jnp.int32),
        pltpu.VMEM((E_CH, 8), jnp.float32),
        pltpu.VMEM((16,), jnp.int32),
        pltpu.VMEM_SHARED((_acc_rows(HALF), 8), jnp.float32),
        pltpu.VMEM_SHARED((_acc_rows(N_G), 8), jnp.float32),
    ],
)


def _cls_body(xp, xg, ls, ld, out,
              sidx0, sidx1, didx0, didx1, rp0, rp1, rg0, rg1,
              semp0, semp1, semg0, semg1, tb, ov):
  c = lax.axis_index("c")
  s = lax.axis_index("s")
  base = (c * NS + s) * L_TILE
  ridx = _iota16() * 16
  n_chunks = L_TILE // L_CH
  sidx = (sidx0, sidx1)
  didx = (didx0, didx1)
  rp = (rp0, rp1)
  rg = (rg0, rg1)
  semp = (semp0, semp1)
  semg = (semg0, semg1)

  def load(k, b):
    cb = pl.multiple_of(base + k * L_CH, 32)
    pltpu.sync_copy(ls.at[pl.ds(cb, L_CH)], sidx[b])
    pltpu.sync_copy(ld.at[pl.ds(cb, L_CH)], didx[b])
    pltpu.async_copy(xp.at[sidx[b]], rp[b], semp[b])
    pltpu.async_copy(xg.at[didx[b]], rg[b], semg[b])

  load(0, 0)
  for k in range(n_chunks):
    b = k % 2
    if k + 1 < n_chunks:
      load(k + 1, 1 - b)
    pltpu.make_async_copy(xp.at[sidx[b]], rp[b], semp[b]).wait()
    pltpu.make_async_copy(xg.at[didx[b]], rg[b], semg[b]).wait()
    cb = pl.multiple_of(base + k * L_CH, 32)

    def g16(g, carry):
      for j in range(16):
        r = g * 16 + j
        acc = None
        for m in range(4):
          a = rp[b][r, pl.ds(16 * m, 16)]
          v = rg[b][r, pl.ds(16 * m, 16)]
          av = a * v
          acc = av if acc is None else acc + av
        tb[pl.ds(j * 16, 16)] = acc
      tot = jnp.zeros((16,), jnp.float32)
      for m in range(16):
        tot = tot + plsc.load_gather(tb, [ridx + m])
      ov[pl.ds(g * 16, 16)] = tot
      return carry

    lax.fori_loop(0, L_CH // 16, g16, 0)
    pltpu.sync_copy(ov, out.at[pl.ds(cb, L_CH)])


_classifier = pl.kernel(
    _cls_body,
    out_type=jax.ShapeDtypeStruct((L_PAD,), jnp.float32),
    mesh=_MESH,
    compiler_params=_SC_PARAMS,
    scratch_types=[
        pltpu.VMEM((L_CH,), jnp.int32),
        pltpu.VMEM((L_CH,), jnp.int32),
        pltpu.VMEM((L_CH,), jnp.int32),
        pltpu.VMEM((L_CH,), jnp.int32),
        pltpu.VMEM((L_CH, D), jnp.float32),
        pltpu.VMEM((L_CH, D), jnp.float32),
        pltpu.VMEM((L_CH, D), jnp.float32),
        pltpu.VMEM((L_CH, D), jnp.float32),
        pltpu.SemaphoreType.DMA,
        pltpu.SemaphoreType.DMA,
        pltpu.SemaphoreType.DMA,
        pltpu.SemaphoreType.DMA,
        pltpu.VMEM((256,), jnp.float32),
        pltpu.VMEM((L_CH,), jnp.float32),
    ],
)


def _init_xg_body(gx, w, b, ge, out):
  acc = lax.dot_general(gx[...], w[...], (((1,), (1,)), ((), ())),
                        preferred_element_type=jnp.float32)
  out[...] = acc + b[...] + ge[...]


def _init_xg(gx, w, b2, ge):
  blk = 1000
  return pl.pallas_call(
      _init_xg_body,
      grid=(N_G // blk,),
      in_specs=[
          pl.BlockSpec((blk, 1000), lambda i: (i, 0)),
          pl.BlockSpec((D, 1000), lambda i: (0, 0)),
          pl.BlockSpec((1, D), lambda i: (0, 0)),
          pl.BlockSpec((blk, D), lambda i: (i, 0)),
      ],
      out_specs=pl.BlockSpec((blk, D), lambda i: (i, 0)),
      out_shape=jax.ShapeDtypeStruct((N_G, D), jnp.float32),
  )(gx, w, b2, ge)


def _make_transform_body(planes, relu):
  def body(x, agg, cnt, wl, wr, b, out):
    a = agg[0]
    n = cnt[0, :, 0:1]
    for p in range(1, planes):
      a = a + agg[p]
      n = n + cnt[p, :, 0:1]
    mean = a / jnp.maximum(n, 1.0)
    o = (lax.dot_general(mean, wl[...], (((1,), (1,)), ((), ())),
                         preferred_element_type=jnp.float32)
         + lax.dot_general(x[...], wr[...], (((1,), (1,)), ((), ())),
                           preferred_element_type=jnp.float32)
         + b[...])
    if relu:
      o = jnp.maximum(o, 0.0)
    out[...] = o
  return body


def _transform(x, agg, cnt, wl, wr, b2, relu, blk):
  planes = agg.shape[0]
  rows = x.shape[0]
  return pl.pallas_call(
      _make_transform_body(planes, relu),
      grid=(rows // blk,),
      in_specs=[
          pl.BlockSpec((blk, D), lambda i: (i, 0)),
          pl.BlockSpec((planes, blk, D), lambda i: (0, i, 0)),
          pl.BlockSpec((planes, blk, 8), lambda i: (0, i, 0)),
          pl.BlockSpec((D, D), lambda i: (0, 0)),
          pl.BlockSpec((D, D), lambda i: (0, 0)),
          pl.BlockSpec((1, D), lambda i: (0, 0)),
      ],
      out_specs=pl.BlockSpec((blk, D), lambda i: (i, 0)),
      out_shape=jax.ShapeDtypeStruct((rows, D), jnp.float32),
  )(x, agg, cnt, wl, wr, b2)


_agg_g = _make_agg_g()


def kernel(protein_n_id, go_term_n_id, go_term_x, e_gp_src, e_gp_dst,
           e_pg_src, e_pg_dst, label_src, label_dst, protein_emb,
           go_term_emb, lin_W, lin_b, Wl, bl, Wr):
  f32 = jnp.float32
  xp = jnp.concatenate(
      [protein_emb, jnp.zeros((P_PAD - N_P, D), f32)], axis=0)
  xg = _init_xg(go_term_x, lin_W, lin_b.reshape(1, D), go_term_emb)

  epad = E_PAD - E
  zpad_i = jnp.zeros((epad,), jnp.int32)
  npad_i = jnp.full((epad,), -1, jnp.int32)
  gp_s = jnp.concatenate([e_gp_src, zpad_i])
  gp_d = jnp.concatenate([e_gp_dst, npad_i])
  tpad_i = N_G + (jnp.arange(epad, dtype=jnp.int32) % 16)
  pg_s = jnp.concatenate([e_pg_src, zpad_i])
  pg_d = jnp.concatenate([e_pg_dst, tpad_i])

  z64_p = jnp.zeros((_acc_rows(HALF), D), f32)
  z64_g = jnp.zeros((_acc_rows(N_G), D), f32)
  z8_p = jnp.zeros((_acc_rows(HALF), 8), f32)
  z8_g = jnp.zeros((_acc_rows(N_G), 8), f32)
  ones8 = jnp.ones((E_CH, 8), f32)

  ps0, pd0, ps1, pd1, pcnts = _partition(gp_s, gp_d)
  cnt_p, cnt_g = _counts_all(pd0, pd1, pg_d, pcnts, z8_p, z8_g, ones8)
  cnt_p = cnt_p.reshape(1, P_PAD, 8)

  for layer in range(3):
    relu = layer < 2
    agg_p = _agg_p_split(xg, ps0, pd0, ps1, pd1, pcnts,
                         z64_p).reshape(1, P_PAD, D)
    agg_g = _agg_g(xp, pg_s, pg_d, z64_g)
    xp = _transform(xp, agg_p, cnt_p, Wl[2 * layer], Wr[2 * layer],
                    bl[2 * layer].reshape(1, D), relu, 512)
    xg = _transform(xg, agg_g, cnt_g, Wl[2 * layer + 1], Wr[2 * layer + 1],
                    bl[2 * layer + 1].reshape(1, D), relu, 1000)

  lpad = L_PAD - L
  ls = jnp.concatenate([label_src, jnp.zeros((lpad,), jnp.int32)])
  ld = jnp.concatenate([label_dst, jnp.zeros((lpad,), jnp.int32)])
  pred = _classifier(xp, xg, ls, ld)
  return pred[:L]
